# baseline (speedup 1.0000x reference)
import functools

import jax
import jax.numpy as jnp
from jax.experimental import pallas as pl
from jax.experimental.pallas import tpu as pltpu

_NBVEC = 8


def _body(x_ref, score_ref, vec_ref, sim_ref):
    x2 = x_ref[0]
    score = score_ref[0]
    n = x2.shape[1]
    iota = jax.lax.broadcasted_iota(jnp.int32, (1, n), 1)
    for i in range(_NBVEC):
        m = jnp.max(score)
        idx = jnp.min(jnp.where(score == m, iota, n))
        onehot = (iota == idx).astype(x2.dtype)
        raw = jnp.sum(x2 * onehot, axis=1, keepdims=True)
        diff = x2 - raw
        d2 = jnp.sum(diff * diff, axis=0, keepdims=True)
        sim = jnp.exp(-jnp.sqrt(d2) * (1.0 / 20.0))
        ssum = jnp.sum(sim)
        wsum = jnp.sum(x2 * sim, axis=1)
        vec_ref[0, i, :] = wsum / ssum
        sim_ref[0, i, :] = sim[0]
        score = (1.0 - sim) * score


def kernel(x, applyUMAP):
    del applyUMAP
    B, C, H, W = x.shape
    n = H * W
    x3 = x.reshape(B, C, n)
    score0 = jax.random.uniform(jax.random.key(1), (B, n), dtype=x.dtype)
    score0 = score0.reshape(B, 1, n)
    vecs, sims = pl.pallas_call(
        _body,
        grid=(B,),
        in_specs=[
            pl.BlockSpec((1, C, n), lambda b: (b, 0, 0)),
            pl.BlockSpec((1, 1, n), lambda b: (b, 0, 0)),
        ],
        out_specs=[
            pl.BlockSpec((1, _NBVEC, C), lambda b: (b, 0, 0)),
            pl.BlockSpec((1, _NBVEC, n), lambda b: (b, 0, 0)),
        ],
        out_shape=[
            jax.ShapeDtypeStruct((B, _NBVEC, C), x.dtype),
            jax.ShapeDtypeStruct((B, _NBVEC, n), x.dtype),
        ],
    )(x3, score0)
    selectedPos = jnp.zeros((B, 1, H, W), dtype=x.dtype)
    return (vecs, sims.reshape(B, _NBVEC, H, W), selectedPos)


# speedup vs baseline: 1.2317x; 1.2317x over previous
import functools

import jax
import jax.numpy as jnp
from jax.experimental import pallas as pl
from jax.experimental.pallas import tpu as pltpu

_NBVEC = 8


def _body(x_ref, score_ref, vec_ref, sim_ref):
    x2 = x_ref[0]
    score = score_ref[0]
    n = x2.shape[1]
    iota = jax.lax.broadcasted_iota(jnp.int32, (1, n), 1)
    n2 = jnp.sum(x2 * x2, axis=0, keepdims=True)
    dn = (((1,), (1,)), ((), ()))
    for i in range(_NBVEC):
        m = jnp.max(score)
        idx = jnp.min(jnp.where(score == m, iota, n))
        onehot = (iota == idx).astype(x2.dtype)
        raw = jax.lax.dot_general(x2, onehot, dn,
                                  preferred_element_type=jnp.float32)
        r2 = jnp.sum(raw * raw)
        dot = jax.lax.dot_general(
            raw.T, x2, (((1,), (0,)), ((), ())),
            preferred_element_type=jnp.float32)
        d2 = jnp.maximum(n2 - 2.0 * dot + r2, 0.0)
        d2 = jnp.where(iota == idx, 0.0, d2)
        sim = jnp.exp(-jnp.sqrt(d2) * (1.0 / 20.0))
        ssum = jnp.sum(sim)
        wsum = jax.lax.dot_general(x2, sim, dn,
                                   preferred_element_type=jnp.float32)
        vec_ref[0, i, :] = wsum[:, 0] / ssum
        sim_ref[0, i, :] = sim[0]
        score = (1.0 - sim) * score


def kernel(x, applyUMAP):
    del applyUMAP
    B, C, H, W = x.shape
    n = H * W
    x3 = x.reshape(B, C, n)
    score0 = jax.random.uniform(jax.random.key(1), (B, n), dtype=x.dtype)
    score0 = score0.reshape(B, 1, n)
    vecs, sims = pl.pallas_call(
        _body,
        grid=(B,),
        in_specs=[
            pl.BlockSpec((1, C, n), lambda b: (b, 0, 0)),
            pl.BlockSpec((1, 1, n), lambda b: (b, 0, 0)),
        ],
        out_specs=[
            pl.BlockSpec((1, _NBVEC, C), lambda b: (b, 0, 0)),
            pl.BlockSpec((1, _NBVEC, n), lambda b: (b, 0, 0)),
        ],
        out_shape=[
            jax.ShapeDtypeStruct((B, _NBVEC, C), x.dtype),
            jax.ShapeDtypeStruct((B, _NBVEC, n), x.dtype),
        ],
    )(x3, score0)
    selectedPos = jnp.zeros((B, 1, H, W), dtype=x.dtype)
    return (vecs, sims.reshape(B, _NBVEC, H, W), selectedPos)
